# Initial kernel scaffold; baseline (speedup 1.0000x reference)
import jax
import jax.numpy as jnp
from jax.experimental import pallas as pl


def kernel(feats, edge_index, W1, b1, W2, b2):
    raise NotImplementedError("write your pallas kernel here")


# speedup vs baseline: 9.6945x; 9.6945x over previous
import jax
import jax.numpy as jnp
from jax import lax
from jax.experimental import pallas as pl
from jax.experimental.pallas import tpu as pltpu
from jax.experimental.pallas import tpu_sc as plsc

N = 10000
E = 320000
D = 128
K = 4

NC = 2
NS = 16
LANES = 16
DH = D // NC

RPT = N // NS
N_PAD = N + LANES
DUMMY = N

CHUNK = 128
N_CHUNKS_STAGE = 20
N_STAGES = 8
EPT = CHUNK * N_CHUNKS_STAGE * N_STAGES
E_PAD = EPT * NS

ROW_CHUNKS = RPT // CHUNK
ROW_REM = RPT % CHUNK


def _rsqrt16(d):
    y = jnp.full((LANES,), 1.0, jnp.float32)
    for k in range(1, 10):
        y = jnp.where(d >= jnp.float32(4.0 ** k),
                      jnp.full((LANES,), 2.0 ** (-k), jnp.float32), y)
    y = y * jnp.float32(0.7)
    for _ in range(6):
        y = y * (1.5 - 0.5 * d * y * y)
    return y


def _sc_body(feats, srcp, dstp, rounds_out,
             s_sh, acc_sh, src_v, dst_v, rows_v, t_v, zeros_v, nrm_v):
    c = lax.axis_index("c")
    tid = lax.axis_index("s")
    row_base = tid * RPT
    col_base = c * DH

    def _fill(i, _):
        for g in range(DH // LANES):
            sl = pl.ds(g * LANES, LANES)
            zeros_v[i, sl] = jnp.zeros((LANES,), jnp.float32)
            rows_v[i, sl] = jnp.full((LANES,), 1.0, jnp.float32)
        return 0
    lax.fori_loop(0, CHUNK, _fill, 0)

    for p in range(ROW_CHUNKS):
        pltpu.sync_copy(zeros_v, acc_sh.at[pl.ds(row_base + p * CHUNK, CHUNK)])
    pltpu.sync_copy(zeros_v.at[pl.ds(0, ROW_REM)],
                    acc_sh.at[pl.ds(row_base + ROW_CHUNKS * CHUNK, ROW_REM)])
    plsc.subcore_barrier()

    def _deg_stage(st, _):
        pltpu.sync_copy(dstp.at[tid, pl.ds(st * N_CHUNKS_STAGE, N_CHUNKS_STAGE)],
                        dst_v)

        def _deg_chunk(j, _):
            pltpu.sync_copy(rows_v, acc_sh.at[dst_v.at[j]], add=True)
            return 0
        lax.fori_loop(0, N_CHUNKS_STAGE, _deg_chunk, 0)
        return 0
    lax.fori_loop(0, N_STAGES, _deg_stage, 0)
    plsc.subcore_barrier()

    def _init_block(nrows, loff):
        goff = row_base + loff
        pltpu.sync_copy(acc_sh.at[pl.ds(goff, nrows)], t_v.at[pl.ds(0, nrows)])
        pltpu.sync_copy(zeros_v.at[pl.ds(0, nrows)],
                        acc_sh.at[pl.ds(goff, nrows)])

        def _nrm_row(i, _):
            d = jnp.maximum(t_v[i, pl.ds(0, LANES)], 1.0)
            nrm_v[loff + i, :] = _rsqrt16(d)
            return 0
        lax.fori_loop(0, nrows, _nrm_row, 0)

        pltpu.sync_copy(feats.at[pl.ds(goff, nrows), pl.ds(col_base, DH)],
                        t_v.at[pl.ds(0, nrows)])

        def _scale_row(i, _):
            n = nrm_v[loff + i, :]
            for g in range(DH // LANES):
                sl = pl.ds(g * LANES, LANES)
                t_v[i, sl] = n * t_v[i, sl]
            return 0
        lax.fori_loop(0, nrows, _scale_row, 0)

        pltpu.sync_copy(t_v.at[pl.ds(0, nrows)], s_sh.at[pl.ds(goff, nrows)])

    for p in range(ROW_CHUNKS):
        _init_block(CHUNK, p * CHUNK)
    _init_block(ROW_REM, ROW_CHUNKS * CHUNK)
    plsc.subcore_barrier()

    def _round(r, _):
        def _stage(st, _):
            base = st * N_CHUNKS_STAGE
            pltpu.sync_copy(srcp.at[tid, pl.ds(base, N_CHUNKS_STAGE)], src_v)
            pltpu.sync_copy(dstp.at[tid, pl.ds(base, N_CHUNKS_STAGE)], dst_v)

            def _chunk(j, _):
                pltpu.sync_copy(s_sh.at[src_v.at[j]], rows_v)
                pltpu.sync_copy(rows_v, acc_sh.at[dst_v.at[j]], add=True)
                return 0
            lax.fori_loop(0, N_CHUNKS_STAGE, _chunk, 0)
            return 0
        lax.fori_loop(0, N_STAGES, _stage, 0)
        plsc.subcore_barrier()

        def _rescale_block(nrows, loff):
            goff = row_base + loff
            pltpu.sync_copy(acc_sh.at[pl.ds(goff, nrows)],
                            t_v.at[pl.ds(0, nrows)])
            pltpu.sync_copy(zeros_v.at[pl.ds(0, nrows)],
                            acc_sh.at[pl.ds(goff, nrows)])

            def _row(i, _):
                n = nrm_v[loff + i, :]
                for g in range(DH // LANES):
                    sl = pl.ds(g * LANES, LANES)
                    x = n * t_v[i, sl]
                    t_v[i, sl] = x
                    rows_v[i, sl] = n * x
                return 0
            lax.fori_loop(0, nrows, _row, 0)

            pltpu.sync_copy(
                t_v.at[pl.ds(0, nrows)],
                rounds_out.at[r, pl.ds(goff, nrows), pl.ds(col_base, DH)])
            pltpu.sync_copy(rows_v.at[pl.ds(0, nrows)],
                            s_sh.at[pl.ds(goff, nrows)])

        for p in range(ROW_CHUNKS):
            _rescale_block(CHUNK, p * CHUNK)
        _rescale_block(ROW_REM, ROW_CHUNKS * CHUNK)
        plsc.subcore_barrier()
        return 0
    lax.fori_loop(0, K, _round, 0)


_sc_grand = pl.kernel(
    _sc_body,
    out_type=jax.ShapeDtypeStruct((K, N, D), jnp.float32),
    mesh=plsc.VectorSubcoreMesh(core_axis_name="c", subcore_axis_name="s"),
    compiler_params=pltpu.CompilerParams(use_tc_tiling_on_sc=False),
    scratch_types=[
        pltpu.VMEM_SHARED((N_PAD, DH), jnp.float32),
        pltpu.VMEM_SHARED((N_PAD, DH), jnp.float32),
        pltpu.VMEM((N_CHUNKS_STAGE, CHUNK), jnp.int32),
        pltpu.VMEM((N_CHUNKS_STAGE, CHUNK), jnp.int32),
        pltpu.VMEM((CHUNK, DH), jnp.float32),
        pltpu.VMEM((CHUNK, DH), jnp.float32),
        pltpu.VMEM((CHUNK, DH), jnp.float32),
        pltpu.VMEM((RPT, LANES), jnp.float32),
    ],
)


def _mlp_body(r_ref, f_ref, w1_ref, b1_ref, w2_ref, b2_ref, o_ref):
    x = f_ref[...]
    for k in range(K):
        x = x + r_ref[k]
    x = x * jnp.float32(1.0 / (K + 1))
    h1 = lax.dot_general(x, w1_ref[...], (((1,), (1,)), ((), ())),
                         precision=lax.Precision.HIGHEST,
                         preferred_element_type=jnp.float32)
    h1 = jnp.maximum(h1 + b1_ref[...], 0.0)
    lo = lax.dot_general(h1, w2_ref[...], (((1,), (1,)), ((), ())),
                         precision=lax.Precision.HIGHEST,
                         preferred_element_type=jnp.float32)
    lo = lo + b2_ref[...]
    m = jnp.max(lo, axis=-1, keepdims=True)
    z = lo - m
    lse = jnp.log(jnp.sum(jnp.exp(z), axis=-1, keepdims=True))
    o_ref[...] = z - lse


_MLP_BLOCK = 1000


def _mlp(rounds, feats, W1, b1, W2, b2):
    n_class = W2.shape[0]
    return pl.pallas_call(
        _mlp_body,
        grid=(N // _MLP_BLOCK,),
        in_specs=[
            pl.BlockSpec((K, _MLP_BLOCK, D), lambda i: (0, i, 0)),
            pl.BlockSpec((_MLP_BLOCK, D), lambda i: (i, 0)),
            pl.BlockSpec((D, D), lambda i: (0, 0)),
            pl.BlockSpec((1, D), lambda i: (0, 0)),
            pl.BlockSpec((n_class, D), lambda i: (0, 0)),
            pl.BlockSpec((1, n_class), lambda i: (0, 0)),
        ],
        out_specs=pl.BlockSpec((_MLP_BLOCK, n_class), lambda i: (i, 0)),
        out_shape=jax.ShapeDtypeStruct((N, n_class), jnp.float32),
    )(rounds, feats, W1, b1.reshape(1, -1), W2, b2.reshape(1, -1))


def kernel(feats, edge_index, W1, b1, W2, b2):
    src = edge_index[0]
    dst = edge_index[1]
    pad = E_PAD - E
    srcp = jnp.concatenate([src, jnp.zeros((pad,), jnp.int32)])
    dstp = jnp.concatenate([dst, jnp.full((pad,), DUMMY, jnp.int32)])
    srcp = srcp.reshape(NS, N_STAGES * N_CHUNKS_STAGE, CHUNK)
    dstp = dstp.reshape(NS, N_STAGES * N_CHUNKS_STAGE, CHUNK)
    rounds = _sc_grand(feats, srcp, dstp)
    return _mlp(rounds, feats, W1, b1, W2, b2)
